# baseline (speedup 1.0000x reference)
import functools
import math

import jax
import jax.numpy as jnp
from jax import lax
from jax.experimental import pallas as pl
from jax.experimental.pallas import tpu as pltpu
from jax.experimental.pallas import tpu_sc as plsc

D_MODEL = 64
SCALE = math.sqrt(D_MODEL)
LANES = 16
NC, NS = 2, 16
NW = NC * NS
R = 8
NBUF = 2


def _sc_embed(x, lut):
    batch, hist = x.shape
    bpw = batch // NW
    n_chunks = bpw // R
    assert batch % NW == 0 and bpw % R == 0 and n_chunks % NBUF == 0

    mesh = plsc.VectorSubcoreMesh(core_axis_name="c", subcore_axis_name="s")

    @functools.partial(
        pl.kernel,
        out_type=jax.ShapeDtypeStruct((batch, hist, D_MODEL), jnp.float32),
        mesh=mesh,
        scratch_types=[
            pltpu.VMEM((bpw, hist), jnp.int32),
            pltpu.VMEM((NBUF, R, hist, D_MODEL), jnp.float32),
            pltpu.SemaphoreType.DMA((NBUF,)),
        ],
        compiler_params=pltpu.CompilerParams(use_tc_tiling_on_sc=False),
    )
    def k(x_hbm, lut_hbm, out_hbm, idx_v, rows_v, gsem):
        wid = lax.axis_index("s") * NC + lax.axis_index("c")
        b0 = wid * bpw
        pltpu.sync_copy(x_hbm.at[pl.ds(b0, bpw)], idx_v)

        def fire_gather(c, b):
            for k in range(R):
                pltpu.async_copy(
                    lut_hbm.at[idx_v.at[c * R + k]],
                    rows_v.at[b, k],
                    gsem.at[b],
                )

        def drain_gather(b):
            pltpu.make_async_copy(
                out_hbm.at[pl.ds(0, R)], rows_v.at[b], gsem.at[b]
            ).wait()

        for b in range(NBUF):
            fire_gather(b, b)

        @pl.loop(0, n_chunks, step=NBUF)
        def _(c0):
            for b in range(NBUF):
                c = c0 + b
                drain_gather(b)

                @pl.loop(0, R)
                def _(r):
                    @pl.loop(0, hist, step=10)
                    def _(h0):
                        for dh in range(10):
                            for j in range(D_MODEL // LANES):
                                sl = (b, r, h0 + dh, pl.ds(j * LANES, LANES))
                                rows_v[sl] = rows_v[sl] * SCALE

                pltpu.sync_copy(
                    rows_v.at[b], out_hbm.at[pl.ds(b0 + c * R, R)]
                )

                @pl.when(c + NBUF < n_chunks)
                def _():
                    fire_gather(c + NBUF, b)

    return k(x, lut)


def kernel(x, lut):
    return _sc_embed(x.astype(jnp.int32), lut)
